# Initial kernel scaffold; baseline (speedup 1.0000x reference)
import jax
import jax.numpy as jnp
from jax.experimental import pallas as pl


def kernel(inputs, initial_hidden_state, adj_mat, W_gate_0, b_gate_0, W_cand_0, b_cand_0, W_gate_1, b_gate_1, W_cand_1, b_cand_1):
    raise NotImplementedError("write your pallas kernel here")


# speedup vs baseline: 7.5671x; 7.5671x over previous
import jax
import jax.numpy as jnp
from jax.experimental import pallas as pl
from jax.experimental.pallas import tpu as pltpu

_N = 325
_B = 32
_T = 12
_DIN = 2
_H = 64
_L = 2
_M = 5
_P = _N * _B // 2
_W = _B * _H


def _mm(a, b):
    return jax.lax.dot_general(a, b, (((1,), (0,)), ((), ())),
                               preferred_element_type=jnp.float32)


def _pair(x):
    return x.reshape(_N, _B // 2, 2 * _H).reshape(_P, 2 * _H)


def _wide(p):
    return p.reshape(_N, _B // 2, 2 * _H).reshape(_N, _W)


def _body(x_ref, hinit_ref, adj_ref, adjT_ref,
          wg0x_ref, wg0h_ref, bg0_ref, wc0x_ref, wc0h_ref, bc0_ref,
          wg1x_ref, wg1h_ref, bg1_ref, wc1x_ref, wc1h_ref, bc1_ref,
          out_seq_ref, out_hid_ref, h0_ref, h1_ref):
    t = pl.program_id(0)

    @pl.when(t == 0)
    def _():
        h0_ref[...] = hinit_ref[0]
        h1_ref[...] = hinit_ref[1]

    adj = adj_ref[...]
    adjT = adjT_ref[...]
    d1 = jnp.sum(adjT, axis=0, keepdims=True)
    d2 = jnp.sum(adj, axis=0, keepdims=True)
    s1 = adjT * jnp.where(d1 > 0, 1.0 / d1, 0.0)
    s2 = adj * jnp.where(d2 > 0, 1.0 / d2, 0.0)

    def chebproj(x, w_ref, acc):
        acc = acc + _mm(_pair(x), w_ref[0])
        x1 = _mm(s1, x)
        acc = acc + _mm(_pair(x1), w_ref[1])
        acc = acc + _mm(_pair(2.0 * _mm(s1, x1) - x), w_ref[2])
        x1 = _mm(s2, x)
        acc = acc + _mm(_pair(x1), w_ref[3])
        acc = acc + _mm(_pair(2.0 * _mm(s2, x1) - x), w_ref[4])
        return acc

    def chebproj2(x, wa_ref, wb_ref):
        a = _mm(_pair(x), wa_ref[0])
        b = _mm(_pair(x), wb_ref[0])
        x1 = _mm(s1, x)
        a = a + _mm(_pair(x1), wa_ref[1])
        b = b + _mm(_pair(x1), wb_ref[1])
        x2 = 2.0 * _mm(s1, x1) - x
        a = a + _mm(_pair(x2), wa_ref[2])
        b = b + _mm(_pair(x2), wb_ref[2])
        x1 = _mm(s2, x)
        a = a + _mm(_pair(x1), wa_ref[3])
        b = b + _mm(_pair(x1), wb_ref[3])
        x2 = 2.0 * _mm(s2, x1) - x
        a = a + _mm(_pair(x2), wa_ref[4])
        b = b + _mm(_pair(x2), wb_ref[4])
        return a, b

    def cell(gx, cx, hw, wgh, bgp, wch, bcp):
        g = jax.nn.sigmoid(chebproj(hw, wgh, gx + bgp[...]))
        r = g[:, :2 * _H]
        u = g[:, 2 * _H:]
        c = jnp.tanh(chebproj(_wide(r * _pair(hw)), wch, cx + bcp[...]))
        hp = _pair(hw)
        return _wide(u * hp + (1.0 - u) * c)

    gx0, cx0 = chebproj2(x_ref[0], wg0x_ref, wc0x_ref)
    h0 = cell(gx0, cx0, h0_ref[...],
              wg0h_ref, bg0_ref, wc0h_ref, bc0_ref)
    h0_ref[...] = h0
    gx1, cx1 = chebproj2(h0, wg1x_ref, wc1x_ref)
    h1 = cell(gx1, cx1, h1_ref[...],
              wg1h_ref, bg1_ref, wc1h_ref, bc1_ref)
    h1_ref[...] = h1
    out_seq_ref[0] = h1
    out_hid_ref[0] = h0
    out_hid_ref[1] = h1


def _split_w(w, dx, out):
    wr = w.reshape(dx + _H, _M, out)
    return wr[:dx].transpose(1, 0, 2), wr[dx:].transpose(1, 0, 2)


def _pad_rows(wx, dx):
    return jnp.pad(wx, ((0, 0), (0, _H - dx), (0, 0)))


def _pair_gate_w(w):
    wr, wu = w[:, :, :_H], w[:, :, _H:]
    z = jnp.zeros_like(wr)
    top = jnp.concatenate([wr, z, wu, z], axis=2)
    bot = jnp.concatenate([z, wr, z, wu], axis=2)
    return jnp.concatenate([top, bot], axis=1)


def _pair_cand_w(w):
    z = jnp.zeros_like(w)
    top = jnp.concatenate([w, z], axis=2)
    bot = jnp.concatenate([z, w], axis=2)
    return jnp.concatenate([top, bot], axis=1)


def kernel(inputs, initial_hidden_state, adj_mat, W_gate_0, b_gate_0,
           W_cand_0, b_cand_0, W_gate_1, b_gate_1, W_cand_1, b_cand_1):
    x_t = inputs.transpose(0, 2, 1, 3)
    x_seq = jnp.pad(x_t, ((0, 0), (0, 0), (0, 0), (0, _H - _DIN))) \
        .reshape(_T, _N, _W)
    hinit = initial_hidden_state.reshape(_L, _B, _N, _H) \
        .transpose(0, 2, 1, 3).reshape(_L, _N, _W)

    wg0x, wg0h = _split_w(W_gate_0, _DIN, 2 * _H)
    wc0x, wc0h = _split_w(W_cand_0, _DIN, _H)
    wg1x, wg1h = _split_w(W_gate_1, _H, 2 * _H)
    wc1x, wc1h = _split_w(W_cand_1, _H, _H)
    wg0xp = _pair_gate_w(_pad_rows(wg0x, _DIN))
    wg0hp = _pair_gate_w(wg0h)
    wc0xp = _pair_cand_w(_pad_rows(wc0x, _DIN))
    wc0hp = _pair_cand_w(wc0h)
    wg1xp = _pair_gate_w(wg1x)
    wg1hp = _pair_gate_w(wg1h)
    wc1xp = _pair_cand_w(wc1x)
    wc1hp = _pair_cand_w(wc1h)

    def pair_bias(b):
        parts = jnp.split(b, b.shape[0] // _H)
        return jnp.concatenate([p for q in parts for p in (q, q)]) \
            .reshape(1, -1)

    full = lambda shape: pl.BlockSpec(shape, lambda t: (0,) * len(shape))
    out_seq, out_hid = pl.pallas_call(
        _body,
        grid=(_T,),
        in_specs=[
            pl.BlockSpec((1, _N, _W), lambda t: (t, 0, 0)),
            full((_L, _N, _W)),
            full((_N, _N)),
            full((_N, _N)),
            full((_M, 2 * _H, 4 * _H)), full((_M, 2 * _H, 4 * _H)),
            full((1, 4 * _H)),
            full((_M, 2 * _H, 2 * _H)), full((_M, 2 * _H, 2 * _H)),
            full((1, 2 * _H)),
            full((_M, 2 * _H, 4 * _H)), full((_M, 2 * _H, 4 * _H)),
            full((1, 4 * _H)),
            full((_M, 2 * _H, 2 * _H)), full((_M, 2 * _H, 2 * _H)),
            full((1, 2 * _H)),
        ],
        out_specs=[
            pl.BlockSpec((1, _N, _W), lambda t: (t, 0, 0)),
            full((_L, _N, _W)),
        ],
        out_shape=[
            jax.ShapeDtypeStruct((_T, _N, _W), jnp.float32),
            jax.ShapeDtypeStruct((_L, _N, _W), jnp.float32),
        ],
        scratch_shapes=[
            pltpu.VMEM((_N, _W), jnp.float32),
            pltpu.VMEM((_N, _W), jnp.float32),
        ],
        compiler_params=pltpu.CompilerParams(
            dimension_semantics=("arbitrary",),
        ),
    )(x_seq, hinit, adj_mat, adj_mat.T,
      wg0xp, wg0hp, pair_bias(b_gate_0), wc0xp, wc0hp, pair_bias(b_cand_0),
      wg1xp, wg1hp, pair_bias(b_gate_1), wc1xp, wc1hp, pair_bias(b_cand_1))

    out_hidden = out_hid.reshape(_L, _N, _B, _H).transpose(0, 2, 1, 3) \
        .reshape(_L, _B, _N * _H)
    out_all = out_seq.reshape(_T, _N, _B, _H).transpose(0, 2, 1, 3) \
        .reshape(_T, _B, _N * _H)
    return (out_hidden, out_all)
